# baseline (speedup 1.0000x reference)
import jax
import jax.numpy as jnp
from jax.experimental import pallas as pl
from jax.experimental.pallas import tpu as pltpu

N = 100000
F = 128
FG = 256
H = 8
C = 16
BLK = 4000
NB = N // BLK
NCH = 2


def _ln(x, scale, bias, eps=1e-5):
    mu = jnp.mean(x, axis=-1, keepdims=True)
    var = jnp.mean((x - mu) * (x - mu), axis=-1, keepdims=True)
    return (x - mu) * jax.lax.rsqrt(var + eps) * scale + bias


def _dot(a, b):
    return jnp.dot(a, b, preferred_element_type=jnp.float32)


def _kernel(view_ref, sp_ref, g_ref,
            ln_g2v_s, ln_g2v_b, W_g2v, b_g2v,
            Wl_v, bl_v, Wr_v, br_v, AE_v, bb_v,
            ln_g2s_s, ln_g2s_b, W_g2s, b_g2s,
            Wl_s, bl_s, Wr_s, br_s, AE_s, bb_s,
            ln_pre_s, ln_pre_b, W_mlp, b_mlp,
            out_ref,
            m_v, s_v, w_v, xr_v, m_s, s_s, w_s, xr_s):
    i = pl.program_id(0)

    @pl.when(i == 0)
    def _init():
        g = g_ref[...]
        gv = jnp.maximum(_ln(g, ln_g2v_s[...], ln_g2v_b[...]), 0.0)
        xv = _dot(gv, W_g2v[...]) + b_g2v[...]
        xr_v[...] = bl_v[...] + _dot(xv, Wr_v[...]) + br_v[...]
        gs = jnp.maximum(_ln(g, ln_g2s_s[...], ln_g2s_b[...]), 0.0)
        xs = _dot(gs, W_g2s[...]) + b_g2s[...]
        xr_s[...] = bl_s[...] + _dot(xs, Wr_s[...]) + br_s[...]
        neg = jnp.full((NCH, F), -jnp.inf, jnp.float32)
        zero = jnp.zeros((NCH, F), jnp.float32)
        m_v[...] = neg
        m_s[...] = neg
        s_v[...] = zero
        s_s[...] = zero
        w_v[...] = zero
        w_s[...] = zero

    CH = BLK // NCH

    def chain(x, Wl, xr, AE, m_ref, s_ref, w_ref, k):
        y = _dot(x, Wl)
        z = y + xr
        e = jnp.maximum(z, 0.2 * z)
        lb = _dot(e, AE)
        m_old = m_ref[k:k + 1, :]
        pb = jnp.exp2(jnp.minimum(lb - m_old, 64.0))
        sum_pb = jnp.sum(pb, axis=0, keepdims=True)
        sum_pbz = jnp.sum(pb * z, axis=0, keepdims=True)
        mx = jnp.max(lb, axis=0, keepdims=True)
        m_new = jnp.maximum(m_old, mx)
        c = jnp.exp2(m_old - m_new)
        fix = jnp.any(mx > m_old + 64.0)

        @pl.when(jnp.logical_not(fix))
        def _commit():
            s_ref[k:k + 1, :] = (s_ref[k:k + 1, :] + sum_pb) * c
            w_ref[k:k + 1, :] = (w_ref[k:k + 1, :] + sum_pbz) * c

        @pl.when(fix)
        def _fixup():
            pb2 = jnp.exp2(lb - m_new)
            s_ref[k:k + 1, :] = s_ref[k:k + 1, :] * c + jnp.sum(
                pb2, axis=0, keepdims=True)
            w_ref[k:k + 1, :] = w_ref[k:k + 1, :] * c + jnp.sum(
                pb2 * z, axis=0, keepdims=True)

        m_ref[k:k + 1, :] = m_new

    AEv = AE_v[...]
    AEs = AE_s[...]
    Wlv = Wl_v[...]
    Wls = Wl_s[...]
    xrv = xr_v[...]
    xrs = xr_s[...]
    for k in range(NCH):
        chain(view_ref[k * CH:(k + 1) * CH, :], Wlv, xrv, AEv,
              m_v, s_v, w_v, k)
        chain(sp_ref[k * CH:(k + 1) * CH, :], Wls, xrs, AEs,
              m_s, s_s, w_s, k)

    @pl.when(i == NB - 1)
    def _fin():
        def merge(m_ref, s_ref, w_ref):
            m = jnp.max(m_ref[...], axis=0, keepdims=True)
            c = jnp.exp2(m_ref[...] - m)
            s = jnp.sum(s_ref[...] * c, axis=0, keepdims=True)
            w = jnp.sum(w_ref[...] * c, axis=0, keepdims=True)
            return s, w

        sv, wv = merge(m_v, s_v, w_v)
        ss, ws = merge(m_s, s_s, w_s)
        v2g = wv / sv - xr_v[...] + bb_v[...]
        s2g = ws / ss - xr_s[...] + bb_s[...]
        x = g_ref[...] + jnp.concatenate([v2g, s2g], axis=1)
        y = jnp.maximum(_ln(x, ln_pre_s[...], ln_pre_b[...]), 0.0)
        y = _dot(y, W_mlp[...]) + b_mlp[...]
        out_ref[...] = x + y


def kernel(view_features, scenepoint_features, prev_global_features,
           ln_g2v_s, ln_g2v_b, W_g2v, b_g2v,
           Wl_v, bl_v, Wr_v, br_v, att_v, bias_v,
           ln_g2s_s, ln_g2s_b, W_g2s, b_g2s,
           Wl_s, bl_s, Wr_s, br_s, att_s, bias_s,
           ln_pre_s, ln_pre_b, W_mlp, b_mlp):
    row = lambda a: a.reshape(1, -1)
    heads = jnp.arange(F) // C
    same = (heads[:, None] == heads[None, :]).astype(jnp.float32)
    log2e = 1.4426950408889634
    AE_v = same * (att_v.reshape(-1)[:, None] * log2e)
    AE_s = same * (att_s.reshape(-1)[:, None] * log2e)
    bb_v = row(bl_v + bias_v)
    bb_s = row(bl_s + bias_s)

    blk = pl.BlockSpec((BLK, F), lambda i: (i, 0))

    def full(shape):
        return pl.BlockSpec(shape, lambda i: (0,) * len(shape))

    ins = [
        view_features, scenepoint_features, prev_global_features,
        row(ln_g2v_s), row(ln_g2v_b), W_g2v, row(b_g2v),
        Wl_v, row(bl_v), Wr_v, row(br_v), AE_v, bb_v,
        row(ln_g2s_s), row(ln_g2s_b), W_g2s, row(b_g2s),
        Wl_s, row(bl_s), Wr_s, row(br_s), AE_s, bb_s,
        row(ln_pre_s), row(ln_pre_b), W_mlp, row(b_mlp),
    ]
    in_specs = [blk, blk] + [full(a.shape) for a in ins[2:]]

    scratch = [
        pltpu.VMEM((NCH, F), jnp.float32), pltpu.VMEM((NCH, F), jnp.float32),
        pltpu.VMEM((NCH, F), jnp.float32), pltpu.VMEM((1, F), jnp.float32),
        pltpu.VMEM((NCH, F), jnp.float32), pltpu.VMEM((NCH, F), jnp.float32),
        pltpu.VMEM((NCH, F), jnp.float32), pltpu.VMEM((1, F), jnp.float32),
    ]

    return pl.pallas_call(
        _kernel,
        grid=(NB,),
        in_specs=in_specs,
        out_specs=full((1, FG)),
        out_shape=jax.ShapeDtypeStruct((1, FG), jnp.float32),
        scratch_shapes=scratch,
        compiler_params=pltpu.CompilerParams(
            dimension_semantics=("arbitrary",)),
    )(*ins)


# speedup vs baseline: 1.1908x; 1.1908x over previous
import jax
import jax.numpy as jnp
from jax.experimental import pallas as pl
from jax.experimental.pallas import tpu as pltpu

N = 100000
F = 128
FG = 256
H = 8
C = 16
BLK = 4000
NB = N // BLK
NCH = 2


def _ln(x, scale, bias, eps=1e-5):
    mu = jnp.mean(x, axis=-1, keepdims=True)
    var = jnp.mean((x - mu) * (x - mu), axis=-1, keepdims=True)
    return (x - mu) * jax.lax.rsqrt(var + eps) * scale + bias


def _dot(a, b):
    return jnp.dot(a, b, preferred_element_type=jnp.float32)


def _kernel(view_ref, sp_ref, g_ref,
            ln_g2v_s, ln_g2v_b, W_g2v, b_g2v,
            Wl_v, bl_v, Wr_v, br_v, AE_v, bb_v,
            ln_g2s_s, ln_g2s_b, W_g2s, b_g2s,
            Wl_s, bl_s, Wr_s, br_s, AE_s, bb_s,
            ln_pre_s, ln_pre_b, W_mlp, b_mlp,
            out_ref,
            m_v, s_v, w_v, xr_v, m_s, s_s, w_s, xr_s):
    i = pl.program_id(0)

    @pl.when(i == 0)
    def _init():
        g = g_ref[...]
        gv = jnp.maximum(_ln(g, ln_g2v_s[...], ln_g2v_b[...]), 0.0)
        xv = _dot(gv, W_g2v[...]) + b_g2v[...]
        xr_v[...] = bl_v[...] + _dot(xv, Wr_v[...]) + br_v[...]
        gs = jnp.maximum(_ln(g, ln_g2s_s[...], ln_g2s_b[...]), 0.0)
        xs = _dot(gs, W_g2s[...]) + b_g2s[...]
        xr_s[...] = bl_s[...] + _dot(xs, Wr_s[...]) + br_s[...]
        neg = jnp.full((NCH, F), -jnp.inf, jnp.float32)
        zero = jnp.zeros((NCH, F), jnp.float32)
        m_v[...] = neg
        m_s[...] = neg
        s_v[...] = zero
        s_s[...] = zero
        w_v[...] = zero
        w_s[...] = zero

    CH = BLK // NCH

    AEv = AE_v[...]
    AEs = AE_s[...]
    Wlv = Wl_v[...]
    Wls = Wl_s[...]
    xrv = xr_v[...]
    xrs = xr_s[...]
    chains = []
    for k in range(NCH):
        chains.append((view_ref[k * CH:(k + 1) * CH, :], Wlv, xrv, AEv,
                       m_v, s_v, w_v, k))
        chains.append((sp_ref[k * CH:(k + 1) * CH, :], Wls, xrs, AEs,
                       m_s, s_s, w_s, k))
    ys = [_dot(x, Wl) for (x, Wl, xr, AE, m, s, w, k) in chains]
    zs = [y + c[2] for y, c in zip(ys, chains)]
    es = [jnp.maximum(z, 0.2 * z) for z in zs]
    lbs = [_dot(e, c[3]) for e, c in zip(es, chains)]
    for z, lb, (x, Wl, xr, AE, m_ref, s_ref, w_ref, k) in zip(zs, lbs, chains):
        m_old = m_ref[k:k + 1, :]
        m_new = jnp.maximum(m_old, jnp.max(lb, axis=0, keepdims=True))
        corr = jnp.exp2(m_old - m_new)
        pb = jnp.exp2(lb - m_new)
        s_ref[k:k + 1, :] = s_ref[k:k + 1, :] * corr + jnp.sum(
            pb, axis=0, keepdims=True)
        w_ref[k:k + 1, :] = w_ref[k:k + 1, :] * corr + jnp.sum(
            pb * z, axis=0, keepdims=True)
        m_ref[k:k + 1, :] = m_new

    @pl.when(i == NB - 1)
    def _fin():
        def merge(m_ref, s_ref, w_ref):
            m = jnp.max(m_ref[...], axis=0, keepdims=True)
            c = jnp.exp2(m_ref[...] - m)
            s = jnp.sum(s_ref[...] * c, axis=0, keepdims=True)
            w = jnp.sum(w_ref[...] * c, axis=0, keepdims=True)
            return s, w

        sv, wv = merge(m_v, s_v, w_v)
        ss, ws = merge(m_s, s_s, w_s)
        v2g = wv / sv - xr_v[...] + bb_v[...]
        s2g = ws / ss - xr_s[...] + bb_s[...]
        x = g_ref[...] + jnp.concatenate([v2g, s2g], axis=1)
        y = jnp.maximum(_ln(x, ln_pre_s[...], ln_pre_b[...]), 0.0)
        y = _dot(y, W_mlp[...]) + b_mlp[...]
        out_ref[...] = x + y


def kernel(view_features, scenepoint_features, prev_global_features,
           ln_g2v_s, ln_g2v_b, W_g2v, b_g2v,
           Wl_v, bl_v, Wr_v, br_v, att_v, bias_v,
           ln_g2s_s, ln_g2s_b, W_g2s, b_g2s,
           Wl_s, bl_s, Wr_s, br_s, att_s, bias_s,
           ln_pre_s, ln_pre_b, W_mlp, b_mlp):
    row = lambda a: a.reshape(1, -1)
    heads = jnp.arange(F) // C
    same = (heads[:, None] == heads[None, :]).astype(jnp.float32)
    log2e = 1.4426950408889634
    AE_v = same * (att_v.reshape(-1)[:, None] * log2e)
    AE_s = same * (att_s.reshape(-1)[:, None] * log2e)
    bb_v = row(bl_v + bias_v)
    bb_s = row(bl_s + bias_s)

    blk = pl.BlockSpec((BLK, F), lambda i: (i, 0))

    def full(shape):
        return pl.BlockSpec(shape, lambda i: (0,) * len(shape))

    ins = [
        view_features, scenepoint_features, prev_global_features,
        row(ln_g2v_s), row(ln_g2v_b), W_g2v, row(b_g2v),
        Wl_v, row(bl_v), Wr_v, row(br_v), AE_v, bb_v,
        row(ln_g2s_s), row(ln_g2s_b), W_g2s, row(b_g2s),
        Wl_s, row(bl_s), Wr_s, row(br_s), AE_s, bb_s,
        row(ln_pre_s), row(ln_pre_b), W_mlp, row(b_mlp),
    ]
    in_specs = [blk, blk] + [full(a.shape) for a in ins[2:]]

    scratch = [
        pltpu.VMEM((NCH, F), jnp.float32), pltpu.VMEM((NCH, F), jnp.float32),
        pltpu.VMEM((NCH, F), jnp.float32), pltpu.VMEM((1, F), jnp.float32),
        pltpu.VMEM((NCH, F), jnp.float32), pltpu.VMEM((NCH, F), jnp.float32),
        pltpu.VMEM((NCH, F), jnp.float32), pltpu.VMEM((1, F), jnp.float32),
    ]

    return pl.pallas_call(
        _kernel,
        grid=(NB,),
        in_specs=in_specs,
        out_specs=full((1, FG)),
        out_shape=jax.ShapeDtypeStruct((1, FG), jnp.float32),
        scratch_shapes=scratch,
        compiler_params=pltpu.CompilerParams(
            dimension_semantics=("arbitrary",)),
    )(*ins)


# speedup vs baseline: 1.3639x; 1.1453x over previous
import jax
import jax.numpy as jnp
from jax.experimental import pallas as pl
from jax.experimental.pallas import tpu as pltpu

N = 100000
F = 128
FG = 256
H = 8
C = 16
BLK = 4000
NB = N // BLK
NCH = 2


def _ln(x, scale, bias, eps=1e-5):
    mu = jnp.mean(x, axis=-1, keepdims=True)
    var = jnp.mean((x - mu) * (x - mu), axis=-1, keepdims=True)
    return (x - mu) * jax.lax.rsqrt(var + eps) * scale + bias


def _dot(a, b):
    return jnp.dot(a, b, preferred_element_type=jnp.float32)


def _kernel(view_ref, sp_ref, g_ref,
            ln_g2v_s, ln_g2v_b, W_g2v, b_g2v,
            Wl_v, bl_v, Wr_v, br_v, AE_v, bb_v,
            ln_g2s_s, ln_g2s_b, W_g2s, b_g2s,
            Wl_s, bl_s, Wr_s, br_s, AE_s, bb_s,
            ln_pre_s, ln_pre_b, W_mlp, b_mlp,
            out_ref,
            m_v, s_v, w_v, xr_v, m_s, s_s, w_s, xr_s):
    i = pl.program_id(0)

    @pl.when(i == 0)
    def _init():
        g = g_ref[...]
        gv = jnp.maximum(_ln(g, ln_g2v_s[...], ln_g2v_b[...]), 0.0)
        xv = _dot(gv, W_g2v[...]) + b_g2v[...]
        xr_v[...] = bl_v[...] + _dot(xv, Wr_v[...]) + br_v[...]
        gs = jnp.maximum(_ln(g, ln_g2s_s[...], ln_g2s_b[...]), 0.0)
        xs = _dot(gs, W_g2s[...]) + b_g2s[...]
        xr_s[...] = bl_s[...] + _dot(xs, Wr_s[...]) + br_s[...]
        neg = jnp.full((NCH, F), -jnp.inf, jnp.float32)
        zero = jnp.zeros((NCH, F), jnp.float32)
        m_v[...] = neg
        m_s[...] = neg
        s_v[...] = zero
        s_s[...] = zero
        w_v[...] = zero
        w_s[...] = zero

    CH = BLK // NCH

    def chain(x, Wl, xr, AE, m_ref, s_ref, w_ref, k):
        y = _dot(x, Wl)
        z = y + xr
        e = jnp.maximum(z, 0.2 * z)
        lb = _dot(e, AE)
        m_old = m_ref[k:k + 1, :]
        m_new = jnp.maximum(m_old, jnp.max(lb, axis=0, keepdims=True))
        corr = jnp.exp2(m_old - m_new)
        pb = jnp.exp2(lb - m_new)
        s_ref[k:k + 1, :] = s_ref[k:k + 1, :] * corr + jnp.sum(
            pb, axis=0, keepdims=True)
        w_ref[k:k + 1, :] = w_ref[k:k + 1, :] * corr + jnp.sum(
            pb * z, axis=0, keepdims=True)
        m_ref[k:k + 1, :] = m_new

    AEv = AE_v[...]
    AEs = AE_s[...]
    Wlv = Wl_v[...]
    Wls = Wl_s[...]
    xrv = xr_v[...]
    xrs = xr_s[...]
    for k in range(NCH):
        chain(view_ref[k * CH:(k + 1) * CH, :], Wlv, xrv, AEv,
              m_v, s_v, w_v, k)
        chain(sp_ref[k * CH:(k + 1) * CH, :], Wls, xrs, AEs,
              m_s, s_s, w_s, k)

    @pl.when(i == NB - 1)
    def _fin():
        def merge(m_ref, s_ref, w_ref):
            m = jnp.max(m_ref[...], axis=0, keepdims=True)
            c = jnp.exp2(m_ref[...] - m)
            s = jnp.sum(s_ref[...] * c, axis=0, keepdims=True)
            w = jnp.sum(w_ref[...] * c, axis=0, keepdims=True)
            return s, w

        sv, wv = merge(m_v, s_v, w_v)
        ss, ws = merge(m_s, s_s, w_s)
        v2g = wv / sv - xr_v[...] + bb_v[...]
        s2g = ws / ss - xr_s[...] + bb_s[...]
        x = g_ref[...] + jnp.concatenate([v2g, s2g], axis=1)
        y = jnp.maximum(_ln(x, ln_pre_s[...], ln_pre_b[...]), 0.0)
        y = _dot(y, W_mlp[...]) + b_mlp[...]
        out_ref[...] = x + y


def kernel(view_features, scenepoint_features, prev_global_features,
           ln_g2v_s, ln_g2v_b, W_g2v, b_g2v,
           Wl_v, bl_v, Wr_v, br_v, att_v, bias_v,
           ln_g2s_s, ln_g2s_b, W_g2s, b_g2s,
           Wl_s, bl_s, Wr_s, br_s, att_s, bias_s,
           ln_pre_s, ln_pre_b, W_mlp, b_mlp):
    row = lambda a: a.reshape(1, -1)
    heads = jnp.arange(F) // C
    same = (heads[:, None] == heads[None, :]).astype(jnp.float32)
    log2e = 1.4426950408889634
    AE_v = same * (att_v.reshape(-1)[:, None] * log2e)
    AE_s = same * (att_s.reshape(-1)[:, None] * log2e)
    bb_v = row(bl_v + bias_v)
    bb_s = row(bl_s + bias_s)

    blk = pl.BlockSpec((BLK, F), lambda i: (i, 0))

    def full(shape):
        return pl.BlockSpec(shape, lambda i: (0,) * len(shape))

    ins = [
        view_features, scenepoint_features, prev_global_features,
        row(ln_g2v_s), row(ln_g2v_b), W_g2v, row(b_g2v),
        Wl_v, row(bl_v), Wr_v, row(br_v), AE_v, bb_v,
        row(ln_g2s_s), row(ln_g2s_b), W_g2s, row(b_g2s),
        Wl_s, row(bl_s), Wr_s, row(br_s), AE_s, bb_s,
        row(ln_pre_s), row(ln_pre_b), W_mlp, row(b_mlp),
    ]
    in_specs = [blk, blk] + [full(a.shape) for a in ins[2:]]

    scratch = [
        pltpu.VMEM((NCH, F), jnp.float32), pltpu.VMEM((NCH, F), jnp.float32),
        pltpu.VMEM((NCH, F), jnp.float32), pltpu.VMEM((1, F), jnp.float32),
        pltpu.VMEM((NCH, F), jnp.float32), pltpu.VMEM((NCH, F), jnp.float32),
        pltpu.VMEM((NCH, F), jnp.float32), pltpu.VMEM((1, F), jnp.float32),
    ]

    return pl.pallas_call(
        _kernel,
        grid=(NB,),
        in_specs=in_specs,
        out_specs=full((1, FG)),
        out_shape=jax.ShapeDtypeStruct((1, FG), jnp.float32),
        scratch_shapes=scratch,
        compiler_params=pltpu.CompilerParams(
            dimension_semantics=("arbitrary",)),
    )(*ins)


# speedup vs baseline: 1.3653x; 1.0011x over previous
import jax
import jax.numpy as jnp
from jax.experimental import pallas as pl
from jax.experimental.pallas import tpu as pltpu

N = 100000
F = 128
FG = 256
H = 8
C = 16
BLK = 4000
NB = N // BLK
NCH = 2


def _ln(x, scale, bias, eps=1e-5):
    mu = jnp.mean(x, axis=-1, keepdims=True)
    var = jnp.mean((x - mu) * (x - mu), axis=-1, keepdims=True)
    return (x - mu) * jax.lax.rsqrt(var + eps) * scale + bias


def _dot(a, b):
    return jnp.dot(a, b, preferred_element_type=jnp.float32)


def _kernel(view_ref, sp_ref, g_ref,
            ln_g2v_s, ln_g2v_b, W_g2v, b_g2v,
            Wl_v, bl_v, Wr_v, br_v, AE_v, bb_v,
            ln_g2s_s, ln_g2s_b, W_g2s, b_g2s,
            Wl_s, bl_s, Wr_s, br_s, AE_s, bb_s,
            ln_pre_s, ln_pre_b, W_mlp, b_mlp,
            out_ref,
            m_v, s_v, w_v, xr_v, m_s, s_s, w_s, xr_s):
    i = pl.program_id(0)

    @pl.when(i == 0)
    def _init():
        g = g_ref[...]
        gv = jnp.maximum(_ln(g, ln_g2v_s[...], ln_g2v_b[...]), 0.0)
        xv = _dot(gv, W_g2v[...]) + b_g2v[...]
        xr_v[...] = bl_v[...] + _dot(xv, Wr_v[...]) + br_v[...]
        gs = jnp.maximum(_ln(g, ln_g2s_s[...], ln_g2s_b[...]), 0.0)
        xs = _dot(gs, W_g2s[...]) + b_g2s[...]
        xr_s[...] = bl_s[...] + _dot(xs, Wr_s[...]) + br_s[...]
        neg = jnp.full((NCH, F), -jnp.inf, jnp.float32)
        zero = jnp.zeros((NCH, F), jnp.float32)
        m_v[...] = neg
        m_s[...] = neg
        s_v[...] = zero
        s_s[...] = zero
        w_v[...] = zero
        w_s[...] = zero

    CH = BLK // NCH

    def chain(x, Wl, xr, AE, m_ref, s_ref, w_ref, k):
        y = _dot(x, Wl)
        z = y + xr
        e = jnp.maximum(z, 0.2 * z)
        lb = _dot(e, AE)
        m_old = m_ref[k:k + 1, :]
        m_new = jnp.maximum(m_old, jnp.max(lb, axis=0, keepdims=True))
        corr = jnp.exp2(m_old - m_new)
        pb = jnp.exp2(lb - m_new)
        s_ref[k:k + 1, :] = s_ref[k:k + 1, :] * corr + jnp.sum(
            pb, axis=0, keepdims=True)
        w_ref[k:k + 1, :] = w_ref[k:k + 1, :] * corr + jnp.sum(
            pb * z, axis=0, keepdims=True)
        m_ref[k:k + 1, :] = m_new

    AEv = AE_v[...]
    AEs = AE_s[...]
    Wlv = Wl_v[...]
    Wls = Wl_s[...]
    xrv = xr_v[...]
    xrs = xr_s[...]
    for k in range(NCH):
        chain(view_ref[k * CH:(k + 1) * CH, :], Wlv, xrv, AEv,
              m_v, s_v, w_v, k)
        chain(sp_ref[k * CH:(k + 1) * CH, :], Wls, xrs, AEs,
              m_s, s_s, w_s, k)

    @pl.when(i == NB - 1)
    def _fin():
        def merge(m_ref, s_ref, w_ref):
            m = jnp.max(m_ref[...], axis=0, keepdims=True)
            c = jnp.exp2(m_ref[...] - m)
            s = jnp.sum(s_ref[...] * c, axis=0, keepdims=True)
            w = jnp.sum(w_ref[...] * c, axis=0, keepdims=True)
            return s, w

        sv, wv = merge(m_v, s_v, w_v)
        ss, ws = merge(m_s, s_s, w_s)
        v2g = wv / sv - xr_v[...] + bb_v[...]
        s2g = ws / ss - xr_s[...] + bb_s[...]
        x = g_ref[...] + jnp.concatenate([v2g, s2g], axis=1)
        y = jnp.maximum(_ln(x, ln_pre_s[...], ln_pre_b[...]), 0.0)
        y = _dot(y, W_mlp[...]) + b_mlp[...]
        out_ref[...] = x + y


def kernel(view_features, scenepoint_features, prev_global_features,
           ln_g2v_s, ln_g2v_b, W_g2v, b_g2v,
           Wl_v, bl_v, Wr_v, br_v, att_v, bias_v,
           ln_g2s_s, ln_g2s_b, W_g2s, b_g2s,
           Wl_s, bl_s, Wr_s, br_s, att_s, bias_s,
           ln_pre_s, ln_pre_b, W_mlp, b_mlp):
    row = lambda a: a.reshape(1, -1)
    heads = jnp.arange(F) // C
    same = (heads[:, None] == heads[None, :]).astype(jnp.float32)
    log2e = 1.4426950408889634
    AE_v = same * (att_v.reshape(-1)[:, None] * log2e)
    AE_s = same * (att_s.reshape(-1)[:, None] * log2e)
    bb_v = row(bl_v + bias_v)
    bb_s = row(bl_s + bias_s)

    blk = pl.BlockSpec((BLK, F), lambda i: (i, 0))

    def full(shape):
        return pl.BlockSpec(shape, lambda i: (0,) * len(shape))

    ins = [
        view_features, scenepoint_features, prev_global_features,
        row(ln_g2v_s), row(ln_g2v_b), W_g2v, row(b_g2v),
        Wl_v, row(bl_v), Wr_v, row(br_v), AE_v, bb_v,
        row(ln_g2s_s), row(ln_g2s_b), W_g2s, row(b_g2s),
        Wl_s, row(bl_s), Wr_s, row(br_s), AE_s, bb_s,
        row(ln_pre_s), row(ln_pre_b), W_mlp, row(b_mlp),
    ]
    in_specs = [blk, blk] + [full(a.shape) for a in ins[2:]]

    scratch = [
        pltpu.VMEM((NCH, F), jnp.float32), pltpu.VMEM((NCH, F), jnp.float32),
        pltpu.VMEM((NCH, F), jnp.float32), pltpu.VMEM((1, F), jnp.float32),
        pltpu.VMEM((NCH, F), jnp.float32), pltpu.VMEM((NCH, F), jnp.float32),
        pltpu.VMEM((NCH, F), jnp.float32), pltpu.VMEM((1, F), jnp.float32),
    ]

    return pl.pallas_call(
        _kernel,
        grid=(NB,),
        in_specs=in_specs,
        out_specs=full((1, FG)),
        out_shape=jax.ShapeDtypeStruct((1, FG), jnp.float32),
        scratch_shapes=scratch,
        compiler_params=pltpu.CompilerParams(
            dimension_semantics=("arbitrary",),
            vmem_limit_bytes=100 * 1024 * 1024),
    )(*ins)
